# baseline (speedup 1.0000x reference)
import functools

import jax
import jax.numpy as jnp
from jax import lax
from jax.experimental import pallas as pl
from jax.experimental.pallas import tpu as pltpu
from jax.experimental.pallas import tpu_sc as plsc


@functools.cache
def _make_gather(V, D, B, L):
    info = plsc.get_sparse_core_info()
    NC, NS = info.num_cores, info.num_subcores
    NW = NC * NS
    assert B % NW == 0
    n_idx = B * L
    rows_per_w = B // NW
    b_per_w = rows_per_w * L
    CR = 16
    CHUNK = CR * L
    assert rows_per_w % CR == 0
    n_chunks = rows_per_w // CR

    mesh = plsc.VectorSubcoreMesh(core_axis_name="c", subcore_axis_name="s")

    @functools.partial(
        pl.kernel,
        mesh=mesh,
        out_type=jax.ShapeDtypeStruct((B, L, D), jnp.float32),
        compiler_params=pltpu.CompilerParams(use_tc_tiling_on_sc=False),
        scratch_types=[
            pltpu.VMEM((rows_per_w, L), jnp.int32),
            pltpu.VMEM((2, CR, L, D), jnp.float32),
            pltpu.SemaphoreType.DMA,
            pltpu.SemaphoreType.DMA,
            pltpu.SemaphoreType.DMA,
            pltpu.SemaphoreType.DMA,
        ],
    )
    def gather_kernel(table_hbm, idx_hbm, out_hbm, idx_v, rows_v, g0, g1, w0, w1):
        wid = lax.axis_index("s") * NC + lax.axis_index("c")
        row_base = wid * rows_per_w
        pltpu.sync_copy(idx_hbm.at[pl.ds(row_base, rows_per_w)], idx_v)
        gsem = (g0, g1)
        wsem = (w0, w1)

        def gather(i, slot):
            handles = []
            for r in range(CR):
                handles.append(
                    pltpu.async_copy(
                        table_hbm.at[idx_v.at[i * CR + r]],
                        rows_v.at[slot, r],
                        gsem[slot],
                    )
                )
            return handles

        g = [gather(0, 0), None]
        w = [None, None]
        for i in range(n_chunks):
            cur, nxt = i % 2, (i + 1) % 2
            if i + 1 < n_chunks:
                if w[nxt] is not None:
                    w[nxt].wait()
                g[nxt] = gather(i + 1, nxt)
            for h in g[cur]:
                h.wait()
            w[cur] = pltpu.async_copy(
                rows_v.at[cur],
                out_hbm.at[pl.ds(row_base + i * CR, CR)],
                wsem[cur],
            )
        for h in w:
            if h is not None:
                h.wait()

    return gather_kernel


def kernel(word_table, indices):
    B, L = indices.shape
    V, D = word_table.shape
    fn = _make_gather(V, D, B, L)
    return fn(word_table, indices)
